# baseline (speedup 1.0000x reference)
import functools

import jax
import jax.numpy as jnp
from jax import lax
from jax.experimental import pallas as pl
from jax.experimental.pallas import tpu as pltpu
from jax.experimental.pallas import tpu_sc as plsc

N = 10000
E = 320000
D = 128
R = 2

NC = 2
NS = 16
NW = NC * NS
EPW = E // NW
CH = 80
NCH = EPW // CH
STRIPE = 632

_mesh = plsc.VectorSubcoreMesh(core_axis_name="c", subcore_axis_name="s")


def _make_sc_dual(rel_b: bool):
    scratch = [
        pltpu.VMEM((EPW,), jnp.int32),
        pltpu.VMEM((NCH, CH), jnp.int32),
        pltpu.VMEM((CH, D), jnp.float32),
        pltpu.VMEM((CH, D), jnp.float32),
        pltpu.VMEM_SHARED((N, D), jnp.float32),
        pltpu.SemaphoreType.DMA,
        pltpu.SemaphoreType.DMA,
        pltpu.SemaphoreType.DMA,
        pltpu.SemaphoreType.DMA,
        pltpu.SemaphoreType.DMA,
    ]

    def body(*refs):
        if rel_b:
            (tab_a, src_a, dst_a, tab_b, src_b, et_b, dst_b, zero_h, out_h,
             gi_v, dst2d, rows_0, rows_1, acc,
             sem_bulk, sem_g0, sem_g1, sem_s0, sem_s1) = refs
        else:
            (tab_a, src_a, dst_a, tab_b, src_b, dst_b, zero_h, out_h,
             gi_v, dst2d, rows_0, rows_1, acc,
             sem_bulk, sem_g0, sem_g1, sem_s0, sem_s1) = refs
        cid = lax.axis_index("c")
        sid = lax.axis_index("s")

        def prologue(src_h, et_h, dst_h, blk):
            base = pl.multiple_of(blk * EPW, 8)
            pltpu.async_copy(src_h.at[pl.ds(base, EPW)], gi_v, sem_bulk).wait()
            if et_h is not None:
                pltpu.sync_copy(et_h.at[blk], dst2d)

                @pl.loop(0, NCH)
                def _(r):
                    for c in range(CH // 16):
                        sl = pl.ds(pl.multiple_of(r * CH + c * 16, 16), 16)
                        gi_v[sl] = gi_v[sl] + dst2d[r, pl.ds(c * 16, 16)] * jnp.int32(N)

            pltpu.sync_copy(dst_h.at[blk], dst2d)

        def main_loop(tab):
            def gather(j, rows, sem):
                off = pl.multiple_of(j * CH, 8)
                pltpu.async_copy(tab.at[gi_v.at[pl.ds(off, CH)]], rows, sem)

            def gwait(rows, sem):
                pltpu.make_async_copy(tab.at[gi_v.at[pl.ds(0, CH)]], rows, sem).wait()

            def scat(j, rows, sem):
                pltpu.async_copy(rows, acc.at[dst2d.at[j]], sem, add=True)

            def swait(rows, sem):
                pltpu.make_async_copy(rows, acc.at[dst2d.at[0]], sem).wait()

            r0, g0, s0 = rows_0, sem_g0, sem_s0
            r1, g1, s1 = rows_1, sem_g1, sem_s1

            assert NCH % 2 == 1
            gather(0, r0, g0)
            gather(1, r1, g1)

            @pl.loop(0, NCH - 3, step=2)
            def _(j):
                gwait(r0, g0)
                scat(j, r0, s0)
                gwait(r1, g1)
                scat(j + 1, r1, s1)
                swait(r0, s0)
                gather(j + 2, r0, g0)
                swait(r1, s1)
                gather(j + 3, r1, g1)

            e = NCH - 3
            gwait(r0, g0)
            scat(e, r0, s0)
            gwait(r1, g1)
            scat(e + 1, r1, s1)
            swait(r0, s0)
            gather(e + 2, r0, g0)
            gwait(r0, g0)
            scat(e + 2, r0, s0)
            swait(r1, s1)
            swait(r0, s0)

        def run(tab, src_h, et_h, dst_h, first: bool):
            if not first:
                prologue(src_h, et_h, dst_h, sid * 2)
            main_loop(tab)
            prologue(src_h, et_h, dst_h, sid * 2 + 1)
            main_loop(tab)

        eb = et_b if rel_b else None

        @pl.when(cid == 0)
        def _():
            prologue(src_a, None, dst_a, sid * 2)

        @pl.when(cid == 1)
        def _():
            prologue(src_b, eb, dst_b, sid * 2)

        @pl.when(sid < NS - 1)
        def _():
            r0 = pl.multiple_of(sid * STRIPE, 8)
            pltpu.sync_copy(zero_h.at[pl.ds(r0, STRIPE)],
                            acc.at[pl.ds(r0, STRIPE)])

        @pl.when(sid == NS - 1)
        def _():
            r0 = (NS - 1) * STRIPE
            pltpu.sync_copy(zero_h.at[pl.ds(r0, N - r0)],
                            acc.at[pl.ds(r0, N - r0)])

        plsc.subcore_barrier()

        @pl.when(cid == 0)
        def _():
            run(tab_a, src_a, None, dst_a, first=True)

        @pl.when(cid == 1)
        def _():
            run(tab_b, src_b, eb, dst_b, first=True)

        plsc.subcore_barrier()

        @pl.when(sid < NS - 1)
        def _():
            r0 = pl.multiple_of(sid * STRIPE, 8)
            pltpu.sync_copy(acc.at[pl.ds(r0, STRIPE)],
                            out_h.at[cid].at[pl.ds(r0, STRIPE)])

        @pl.when(sid == NS - 1)
        def _():
            r0 = (NS - 1) * STRIPE
            pltpu.sync_copy(acc.at[pl.ds(r0, N - r0)],
                            out_h.at[cid].at[pl.ds(r0, N - r0)])

    return functools.partial(
        pl.kernel, mesh=_mesh,
        out_type=jax.ShapeDtypeStruct((NC, N, D), jnp.float32),
        scratch_types=scratch,
    )(body)


_sc_dual = _make_sc_dual(rel_b=False)
_sc_dual_rel = _make_sc_dual(rel_b=True)


def _dot(a, b):
    return lax.dot_general(a, b, (((1,), (0,)), ((), ())),
                           preferred_element_type=jnp.float32,
                           precision=lax.Precision.DEFAULT)


BN = 2000
_GRID = (N // BN,)

_b_rows = pl.BlockSpec((BN, D), lambda i: (i, 0))
_b_parts = pl.BlockSpec((NC, BN, D), lambda i: (0, i, 0))
_b_w = pl.BlockSpec((D, D), lambda i: (0, 0))
_b_wr = pl.BlockSpec((R, D, D), lambda i: (0, 0, 0))
_b_bias = pl.BlockSpec((1, D), lambda i: (0, 0))
_b_trows = pl.BlockSpec((R, BN, D), lambda i: (0, i, 0))


def _tc_slot1(parts, W_a, b_a, W_b, b_b):
    def body(p_ref, wa_ref, ba_ref, wb_ref, bb_ref, h_ref, g_ref):
        h_ref[...] = jnp.tanh(_dot(p_ref[0], wa_ref[...]) + ba_ref[...])
        g_ref[...] = jnp.tanh(_dot(p_ref[1], wb_ref[...]) + bb_ref[...])
    return pl.pallas_call(
        body, grid=_GRID,
        in_specs=[_b_parts, _b_w, _b_bias, _b_w, _b_bias],
        out_specs=(_b_rows, _b_rows),
        out_shape=(jax.ShapeDtypeStruct((N, D), jnp.float32),
                   jax.ShapeDtypeStruct((N, D), jnp.float32)),
    )(parts, W_a, b_a.reshape(1, D), W_b, b_b.reshape(1, D))


def _tc_slot2(parts, W_g2, b_g2, S1, b_r1, W_rel2, W_self2):
    def body(p_ref, wg_ref, bg_ref, s1_ref, br_ref, wr_ref, ws_ref,
             hcf_ref, t_ref, s2_ref):
        hcf_ref[...] = jnp.tanh(_dot(p_ref[0], wg_ref[...]) + bg_ref[...])
        r = jnp.tanh(p_ref[1] + s1_ref[...] + br_ref[...])
        t_ref[0] = _dot(r, wr_ref[0])
        t_ref[1] = _dot(r, wr_ref[1])
        s2_ref[...] = _dot(r, ws_ref[...])
    return pl.pallas_call(
        body, grid=_GRID,
        in_specs=[_b_parts, _b_w, _b_bias, _b_rows, _b_bias, _b_wr, _b_w],
        out_specs=(_b_rows, _b_trows, _b_rows),
        out_shape=(jax.ShapeDtypeStruct((N, D), jnp.float32),
                   jax.ShapeDtypeStruct((R, N, D), jnp.float32),
                   jax.ShapeDtypeStruct((N, D), jnp.float32)),
    )(parts, W_g2, b_g2.reshape(1, D), S1, b_r1.reshape(1, D), W_rel2, W_self2)


def _tc_slot3(parts, W_h2, b_h2, S2, b_r2):
    def body(p_ref, wh_ref, bh_ref, s2_ref, br_ref, hs_ref, hc_ref):
        hs_ref[...] = jnp.tanh(_dot(p_ref[0], wh_ref[...]) + bh_ref[...])
        hc_ref[...] = jnp.tanh(p_ref[1] + s2_ref[...] + br_ref[...])
    return pl.pallas_call(
        body, grid=_GRID,
        in_specs=[_b_parts, _b_w, _b_bias, _b_rows, _b_bias],
        out_specs=(_b_rows, _b_rows),
        out_shape=(jax.ShapeDtypeStruct((N, D), jnp.float32),
                   jax.ShapeDtypeStruct((N, D), jnp.float32)),
    )(parts, W_h2, b_h2.reshape(1, D), S2, b_r2.reshape(1, D))


def _tc_rgcn_prep(x, W_rel, W_self):
    def body(x_ref, wr_ref, ws_ref, t_ref, s_ref):
        xv = x_ref[...]
        t_ref[0] = _dot(xv, wr_ref[0])
        t_ref[1] = _dot(xv, wr_ref[1])
        s_ref[...] = _dot(xv, ws_ref[...])
    return pl.pallas_call(
        body, grid=_GRID,
        in_specs=[_b_rows, _b_wr, _b_w],
        out_specs=(_b_trows, _b_rows),
        out_shape=(jax.ShapeDtypeStruct((R, N, D), jnp.float32),
                   jax.ShapeDtypeStruct((N, D), jnp.float32)),
    )(x, W_rel, W_self)


def kernel(node_ids, edge_index_l, edge_index_bi, etypes, edge_index_h,
           emb_table, W_g1, b_g1, W_g2, b_g2,
           W_rel1, W_self1, b_r1, W_rel2, W_self2, b_r2,
           W_h1, b_h1, W_h2, b_h2):
    emb = emb_table
    zeros = jnp.zeros((N, D), jnp.float32)
    src_l, dst_l = edge_index_l[0], edge_index_l[1].reshape(NW, NCH, CH)
    src_b, dst_b = edge_index_bi[0], edge_index_bi[1].reshape(NW, NCH, CH)
    src_h, dst_h = edge_index_h[0], edge_index_h[1].reshape(NW, NCH, CH)
    et3 = etypes.reshape(NW, NCH, CH)

    o1 = _sc_dual(emb, src_l, dst_l, emb, src_h, dst_h, zeros)
    T1, S1 = _tc_rgcn_prep(emb, W_rel1, W_self1)
    h1, g1 = _tc_slot1(o1, W_g1, b_g1, W_h1, b_h1)

    o2 = _sc_dual_rel(h1, src_l, dst_l, T1.reshape(R * N, D),
                      src_b, et3, dst_b, zeros)
    hcf, T2, S2 = _tc_slot2(o2, W_g2, b_g2, S1, b_r1, W_rel2, W_self2)

    o3 = _sc_dual_rel(g1, src_h, dst_h, T2.reshape(R * N, D),
                      src_b, et3, dst_b, zeros)
    hs, hc = _tc_slot3(o3, W_h2, b_h2, S2, b_r2)
    return (hcf, hc, hs)


# speedup vs baseline: 1.2737x; 1.2737x over previous
import functools

import jax
import jax.numpy as jnp
from jax import lax
from jax.experimental import pallas as pl
from jax.experimental.pallas import tpu as pltpu
from jax.experimental.pallas import tpu_sc as plsc

N = 10000
E = 320000
D = 128
R = 2

NC = 2
NS = 16
NW = NC * NS
EPW = E // NW
CH = 80
NCH = EPW // CH
STRIPE = 632

_mesh = plsc.VectorSubcoreMesh(core_axis_name="c", subcore_axis_name="s")


def _make_sc_dual(rel_b: bool):
    scratch = [
        pltpu.VMEM((EPW,), jnp.int32),
        pltpu.VMEM((NCH, CH), jnp.int32),
        pltpu.VMEM((CH, D), jnp.float32),
        pltpu.VMEM((CH, D), jnp.float32),
        pltpu.VMEM_SHARED((N, D), jnp.float32),
        pltpu.SemaphoreType.DMA,
        pltpu.SemaphoreType.DMA,
        pltpu.SemaphoreType.DMA,
    ]

    def body(*refs):
        if rel_b:
            (tab_a, src_a, dst_a, tab_b, src_b, et_b, dst_b, zero_h, out_h,
             gi_v, dst2d, rows_0, rows_1, acc,
             sem_bulk, sem_g0, sem_g1) = refs
        else:
            (tab_a, src_a, dst_a, tab_b, src_b, dst_b, zero_h, out_h,
             gi_v, dst2d, rows_0, rows_1, acc,
             sem_bulk, sem_g0, sem_g1) = refs
        cid = lax.axis_index("c")
        sid = lax.axis_index("s")

        def prologue(src_h, et_h, dst_h, blk):
            base = pl.multiple_of(blk * EPW, 8)
            pltpu.async_copy(src_h.at[pl.ds(base, EPW)], gi_v, sem_bulk).wait()
            if et_h is not None:
                pltpu.sync_copy(et_h.at[blk], dst2d)

                @pl.loop(0, NCH)
                def _(r):
                    for c in range(CH // 16):
                        sl = pl.ds(pl.multiple_of(r * CH + c * 16, 16), 16)
                        gi_v[sl] = gi_v[sl] + dst2d[r, pl.ds(c * 16, 16)] * jnp.int32(N)

            pltpu.sync_copy(dst_h.at[blk], dst2d)

        def main_loop(tab):
            def gather(j, rows, sem):
                off = pl.multiple_of(j * CH, 8)
                pltpu.async_copy(tab.at[gi_v.at[pl.ds(off, CH)]], rows, sem)

            def gwait(rows, sem):
                pltpu.make_async_copy(tab.at[gi_v.at[pl.ds(0, CH)]], rows, sem).wait()

            def scatter(rows, j):
                pltpu.sync_copy(rows, acc.at[dst2d.at[j]], add=True)

            gather(0, rows_0, sem_g0)

            @pl.loop(0, NCH - 1, step=2)
            def _(j):
                gather(j + 1, rows_1, sem_g1)
                gwait(rows_0, sem_g0)
                scatter(rows_0, j)
                gather(j + 2, rows_0, sem_g0)
                gwait(rows_1, sem_g1)
                scatter(rows_1, j + 1)

            gwait(rows_0, sem_g0)
            scatter(rows_0, NCH - 1)

        def run(tab, src_h, et_h, dst_h, first: bool):
            if not first:
                prologue(src_h, et_h, dst_h, sid * 2)
            main_loop(tab)
            prologue(src_h, et_h, dst_h, sid * 2 + 1)
            main_loop(tab)

        eb = et_b if rel_b else None

        @pl.when(cid == 0)
        def _():
            prologue(src_a, None, dst_a, sid * 2)

        @pl.when(cid == 1)
        def _():
            prologue(src_b, eb, dst_b, sid * 2)

        @pl.when(sid < NS - 1)
        def _():
            r0 = pl.multiple_of(sid * STRIPE, 8)
            pltpu.sync_copy(zero_h.at[pl.ds(r0, STRIPE)],
                            acc.at[pl.ds(r0, STRIPE)])

        @pl.when(sid == NS - 1)
        def _():
            r0 = (NS - 1) * STRIPE
            pltpu.sync_copy(zero_h.at[pl.ds(r0, N - r0)],
                            acc.at[pl.ds(r0, N - r0)])

        plsc.subcore_barrier()

        @pl.when(cid == 0)
        def _():
            run(tab_a, src_a, None, dst_a, first=True)

        @pl.when(cid == 1)
        def _():
            run(tab_b, src_b, eb, dst_b, first=True)

        plsc.subcore_barrier()

        @pl.when(sid < NS - 1)
        def _():
            r0 = pl.multiple_of(sid * STRIPE, 8)
            pltpu.sync_copy(acc.at[pl.ds(r0, STRIPE)],
                            out_h.at[cid].at[pl.ds(r0, STRIPE)])

        @pl.when(sid == NS - 1)
        def _():
            r0 = (NS - 1) * STRIPE
            pltpu.sync_copy(acc.at[pl.ds(r0, N - r0)],
                            out_h.at[cid].at[pl.ds(r0, N - r0)])

    return functools.partial(
        pl.kernel, mesh=_mesh,
        out_type=jax.ShapeDtypeStruct((NC, N, D), jnp.float32),
        scratch_types=scratch,
    )(body)


_sc_dual = _make_sc_dual(rel_b=False)
_sc_dual_rel = _make_sc_dual(rel_b=True)


def _dot(a, b):
    return lax.dot_general(a, b, (((1,), (0,)), ((), ())),
                           preferred_element_type=jnp.float32,
                           precision=lax.Precision.DEFAULT)


BN = 2000
_GRID = (N // BN,)

_b_rows = pl.BlockSpec((BN, D), lambda i: (i, 0))
_b_parts = pl.BlockSpec((NC, BN, D), lambda i: (0, i, 0))
_b_w = pl.BlockSpec((D, D), lambda i: (0, 0))
_b_wr = pl.BlockSpec((R, D, D), lambda i: (0, 0, 0))
_b_bias = pl.BlockSpec((1, D), lambda i: (0, 0))
_b_trows = pl.BlockSpec((R, BN, D), lambda i: (0, i, 0))


def _tc_slot1(parts, W_a, b_a, W_b, b_b):
    def body(p_ref, wa_ref, ba_ref, wb_ref, bb_ref, h_ref, g_ref):
        h_ref[...] = jnp.tanh(_dot(p_ref[0], wa_ref[...]) + ba_ref[...])
        g_ref[...] = jnp.tanh(_dot(p_ref[1], wb_ref[...]) + bb_ref[...])
    return pl.pallas_call(
        body, grid=_GRID,
        in_specs=[_b_parts, _b_w, _b_bias, _b_w, _b_bias],
        out_specs=(_b_rows, _b_rows),
        out_shape=(jax.ShapeDtypeStruct((N, D), jnp.float32),
                   jax.ShapeDtypeStruct((N, D), jnp.float32)),
    )(parts, W_a, b_a.reshape(1, D), W_b, b_b.reshape(1, D))


def _tc_slot2(parts, W_g2, b_g2, S1, b_r1, W_rel2, W_self2):
    def body(p_ref, wg_ref, bg_ref, s1_ref, br_ref, wr_ref, ws_ref,
             hcf_ref, t_ref, s2_ref):
        hcf_ref[...] = jnp.tanh(_dot(p_ref[0], wg_ref[...]) + bg_ref[...])
        r = jnp.tanh(p_ref[1] + s1_ref[...] + br_ref[...])
        t_ref[0] = _dot(r, wr_ref[0])
        t_ref[1] = _dot(r, wr_ref[1])
        s2_ref[...] = _dot(r, ws_ref[...])
    return pl.pallas_call(
        body, grid=_GRID,
        in_specs=[_b_parts, _b_w, _b_bias, _b_rows, _b_bias, _b_wr, _b_w],
        out_specs=(_b_rows, _b_trows, _b_rows),
        out_shape=(jax.ShapeDtypeStruct((N, D), jnp.float32),
                   jax.ShapeDtypeStruct((R, N, D), jnp.float32),
                   jax.ShapeDtypeStruct((N, D), jnp.float32)),
    )(parts, W_g2, b_g2.reshape(1, D), S1, b_r1.reshape(1, D), W_rel2, W_self2)


def _tc_slot3(parts, W_h2, b_h2, S2, b_r2):
    def body(p_ref, wh_ref, bh_ref, s2_ref, br_ref, hs_ref, hc_ref):
        hs_ref[...] = jnp.tanh(_dot(p_ref[0], wh_ref[...]) + bh_ref[...])
        hc_ref[...] = jnp.tanh(p_ref[1] + s2_ref[...] + br_ref[...])
    return pl.pallas_call(
        body, grid=_GRID,
        in_specs=[_b_parts, _b_w, _b_bias, _b_rows, _b_bias],
        out_specs=(_b_rows, _b_rows),
        out_shape=(jax.ShapeDtypeStruct((N, D), jnp.float32),
                   jax.ShapeDtypeStruct((N, D), jnp.float32)),
    )(parts, W_h2, b_h2.reshape(1, D), S2, b_r2.reshape(1, D))


def _tc_rgcn_prep(x, W_rel, W_self):
    def body(x_ref, wr_ref, ws_ref, t_ref, s_ref):
        xv = x_ref[...]
        t_ref[0] = _dot(xv, wr_ref[0])
        t_ref[1] = _dot(xv, wr_ref[1])
        s_ref[...] = _dot(xv, ws_ref[...])
    return pl.pallas_call(
        body, grid=_GRID,
        in_specs=[_b_rows, _b_wr, _b_w],
        out_specs=(_b_trows, _b_rows),
        out_shape=(jax.ShapeDtypeStruct((R, N, D), jnp.float32),
                   jax.ShapeDtypeStruct((N, D), jnp.float32)),
    )(x, W_rel, W_self)


def kernel(node_ids, edge_index_l, edge_index_bi, etypes, edge_index_h,
           emb_table, W_g1, b_g1, W_g2, b_g2,
           W_rel1, W_self1, b_r1, W_rel2, W_self2, b_r2,
           W_h1, b_h1, W_h2, b_h2):
    emb = emb_table
    zeros = jnp.zeros((N, D), jnp.float32)
    src_l, dst_l = edge_index_l[0], edge_index_l[1].reshape(NW, NCH, CH)
    src_b, dst_b = edge_index_bi[0], edge_index_bi[1].reshape(NW, NCH, CH)
    src_h, dst_h = edge_index_h[0], edge_index_h[1].reshape(NW, NCH, CH)
    et3 = etypes.reshape(NW, NCH, CH)

    o1 = _sc_dual(emb, src_l, dst_l, emb, src_h, dst_h, zeros)
    T1, S1 = _tc_rgcn_prep(emb, W_rel1, W_self1)
    h1, g1 = _tc_slot1(o1, W_g1, b_g1, W_h1, b_h1)

    o2 = _sc_dual_rel(h1, src_l, dst_l, T1.reshape(R * N, D),
                      src_b, et3, dst_b, zeros)
    hcf, T2, S2 = _tc_slot2(o2, W_g2, b_g2, S1, b_r1, W_rel2, W_self2)

    o3 = _sc_dual_rel(g1, src_h, dst_h, T2.reshape(R * N, D),
                      src_b, et3, dst_b, zeros)
    hs, hc = _tc_slot3(o3, W_h2, b_h2, S2, b_r2)
    return (hcf, hc, hs)
